# Initial kernel scaffold; baseline (speedup 1.0000x reference)
import jax
import jax.numpy as jnp
from jax.experimental import pallas as pl


def kernel(edge_attr, W_bond, W_stereo, W_conj):
    raise NotImplementedError("write your pallas kernel here")


# speedup vs baseline: 1.2597x; 1.2597x over previous
import functools

import jax
import jax.numpy as jnp
from jax import lax
from jax.experimental import pallas as pl
from jax.experimental.pallas import tpu as pltpu
from jax.experimental.pallas import tpu_sc as plsc

E = 160000
H = 256
CHUNK = 128
LANES = 16


def _sc_body(e0, e1, e2, table, out, idx0_v, idx1_v, idx2_v, comb_v, rows_v, sem):
    info = plsc.get_sparse_core_info()
    nc, ns = info.num_cores, info.num_subcores
    nw = nc * ns
    wid = lax.axis_index("s") * nc + lax.axis_index("c")
    n_chunks = E // CHUNK
    iters = (n_chunks + nw - 1) // nw

    def body(k, carry):
        chunk = k * nw + wid

        @pl.when(chunk < n_chunks)
        def _():
            off = chunk * CHUNK
            pltpu.sync_copy(e0.at[pl.ds(off, CHUNK)], idx0_v)
            pltpu.sync_copy(e1.at[pl.ds(off, CHUNK)], idx1_v)
            pltpu.sync_copy(e2.at[pl.ds(off, CHUNK)], idx2_v)
            for w in range(CHUNK // LANES):
                s = pl.ds(w * LANES, LANES)
                comb_v[s] = idx0_v[s] * 12 + idx1_v[s] * 2 + idx2_v[s]
            pltpu.async_copy(table.at[comb_v], rows_v, sem).wait()
            pltpu.sync_copy(rows_v, out.at[pl.ds(off, CHUNK)])

        return carry

    lax.fori_loop(0, iters, body, 0)


def kernel(edge_attr, W_bond, W_stereo, W_conj):
    table = (W_bond[:, None, None, :]
             + W_stereo[None, :, None, :]
             + W_conj[None, None, :, :]).reshape(276, H)
    ea = edge_attr.astype(jnp.int32)
    e0, e1, e2 = ea[:, 0], ea[:, 1], ea[:, 2]
    mesh = plsc.VectorSubcoreMesh(core_axis_name="c", subcore_axis_name="s")
    run = functools.partial(
        pl.kernel,
        mesh=mesh,
        out_type=jax.ShapeDtypeStruct((E, H), jnp.float32),
        scratch_types=[
            pltpu.VMEM((CHUNK,), jnp.int32),
            pltpu.VMEM((CHUNK,), jnp.int32),
            pltpu.VMEM((CHUNK,), jnp.int32),
            pltpu.VMEM((CHUNK,), jnp.int32),
            pltpu.VMEM((CHUNK, H), jnp.float32),
            pltpu.SemaphoreType.DMA,
        ],
    )(_sc_body)
    return run(e0, e1, e2, table)


# speedup vs baseline: 1.2602x; 1.0003x over previous
import functools

import jax
import jax.numpy as jnp
from jax import lax
from jax.experimental import pallas as pl
from jax.experimental.pallas import tpu as pltpu
from jax.experimental.pallas import tpu_sc as plsc

E = 160000
H = 256
LANES = 16
NW = 32
SPAN = E // NW
CHUNK = 104
NBUF = 4
NFULL = SPAN // CHUNK
TAIL = SPAN - NFULL * CHUNK
PAD = -(-SPAN // LANES) * LANES


def _sc_body(e0, e1, e2, table, out,
             comb_v, tmp_v, bufs, tail_v,
             g0, g1, g2, g3, w0, w1, w2, w3):
    gsem = (g0, g1, g2, g3)
    wsem = (w0, w1, w2, w3)
    info = plsc.get_sparse_core_info()
    nc = info.num_cores
    wid = lax.axis_index("s") * nc + lax.axis_index("c")
    base = wid * SPAN

    pltpu.sync_copy(e0.at[pl.ds(base, SPAN)], comb_v.at[pl.ds(0, SPAN)])
    pltpu.sync_copy(e1.at[pl.ds(base, SPAN)], tmp_v.at[pl.ds(0, SPAN)])

    def fold1(i, c):
        s = pl.ds(i * LANES, LANES)
        comb_v[s] = comb_v[s] * 12 + tmp_v[s] * 2
        return c

    lax.fori_loop(0, PAD // LANES, fold1, 0)
    pltpu.sync_copy(e2.at[pl.ds(base, SPAN)], tmp_v.at[pl.ds(0, SPAN)])

    def fold2(i, c):
        s = pl.ds(i * LANES, LANES)
        comb_v[s] = comb_v[s] + tmp_v[s]
        return c

    lax.fori_loop(0, PAD // LANES, fold2, 0)

    def start_gather(k, b):
        pltpu.async_copy(table.at[comb_v.at[pl.ds(k * CHUNK, CHUNK)]],
                         bufs.at[b], gsem[b])

    def drain(sem):
        pltpu.make_async_copy(out.at[pl.ds(0, CHUNK)], bufs.at[0], sem).wait()

    for b in range(NBUF):
        start_gather(b, b)

    def step(j, c):
        for b in range(NBUF):
            k = j * NBUF + b
            drain(gsem[b])
            pltpu.async_copy(bufs.at[b], out.at[pl.ds(base + k * CHUNK, CHUNK)],
                             wsem[b])

            @pl.when(j < NFULL // NBUF - 1)
            def _():
                drain(wsem[b])
                start_gather(k + NBUF, b)

        return c

    lax.fori_loop(0, NFULL // NBUF, step, 0)
    for b in range(NBUF):
        drain(wsem[b])

    off = NFULL * CHUNK
    cp = pltpu.async_copy(table.at[comb_v.at[pl.ds(off, TAIL)]], tail_v, g0)
    cp.wait()
    pltpu.sync_copy(tail_v, out.at[pl.ds(base + off, TAIL)])


def kernel(edge_attr, W_bond, W_stereo, W_conj):
    table = (W_bond[:, None, None, :]
             + W_stereo[None, :, None, :]
             + W_conj[None, None, :, :]).reshape(276, H)
    ea = edge_attr.astype(jnp.int32)
    e0, e1, e2 = ea[:, 0], ea[:, 1], ea[:, 2]
    mesh = plsc.VectorSubcoreMesh(core_axis_name="c", subcore_axis_name="s")
    run = functools.partial(
        pl.kernel,
        mesh=mesh,
        out_type=jax.ShapeDtypeStruct((E, H), jnp.float32),
        scratch_types=[
            pltpu.VMEM((PAD,), jnp.int32),
            pltpu.VMEM((PAD,), jnp.int32),
            pltpu.VMEM((NBUF, CHUNK, H), jnp.float32),
            pltpu.VMEM((TAIL, H), jnp.float32),
        ] + [pltpu.SemaphoreType.DMA] * (2 * NBUF),
    )(_sc_body)
    return run(e0, e1, e2, table)


# speedup vs baseline: 4.2626x; 3.3826x over previous
import functools

import jax
import jax.numpy as jnp
from jax import lax
from jax.experimental import pallas as pl
from jax.experimental.pallas import tpu as pltpu
from jax.experimental.pallas import tpu_sc as plsc

E = 160000
H = 256
LANES = 16
NW = 32
SPAN = E // NW
CHUNK = 104
NBUF = 4
NFULL = SPAN // CHUNK
TAIL = SPAN - NFULL * CHUNK
PAD = -(-SPAN // LANES) * LANES


def _sc_body(e0, e1, e2, table, out,
             comb_v, tmp_v, bufs, tail_v,
             g0, g1, g2, g3, w0, w1, w2, w3):
    gsem = (g0, g1, g2, g3)
    wsem = (w0, w1, w2, w3)
    info = plsc.get_sparse_core_info()
    nc = info.num_cores
    sid = lax.axis_index("s")
    wid = sid * nc + lax.axis_index("c")
    base = wid * SPAN


    pltpu.sync_copy(e0.at[pl.ds(base, SPAN)], comb_v.at[pl.ds(0, SPAN)])
    pltpu.sync_copy(e1.at[pl.ds(base, SPAN)], tmp_v.at[pl.ds(0, SPAN)])

    def fold1(i, c):
        s = pl.ds(i * LANES, LANES)
        comb_v[s] = comb_v[s] * 12 + tmp_v[s] * 2
        return c

    lax.fori_loop(0, PAD // LANES, fold1, 0)
    pltpu.sync_copy(e2.at[pl.ds(base, SPAN)], tmp_v.at[pl.ds(0, SPAN)])

    rep_off = wid * 276

    def fold2(i, c):
        s = pl.ds(i * LANES, LANES)
        comb_v[s] = comb_v[s] + tmp_v[s] + rep_off
        return c

    lax.fori_loop(0, PAD // LANES, fold2, 0)

    def start_gather(k, b):
        pltpu.async_copy(table.at[comb_v.at[pl.ds(k * CHUNK, CHUNK)]],
                         bufs.at[b], gsem[b])

    def drain(sem):
        pltpu.make_async_copy(out.at[pl.ds(0, CHUNK)], bufs.at[0], sem).wait()

    for b in range(NBUF):
        start_gather(b, b)

    def step(j, c):
        for b in range(NBUF):
            k = j * NBUF + b
            drain(gsem[b])
            pltpu.async_copy(bufs.at[b], out.at[pl.ds(base + k * CHUNK, CHUNK)],
                             wsem[b])

            @pl.when(j < NFULL // NBUF - 1)
            def _():
                drain(wsem[b])
                start_gather(k + NBUF, b)

        return c

    lax.fori_loop(0, NFULL // NBUF, step, 0)
    for b in range(NBUF):
        drain(wsem[b])

    off = NFULL * CHUNK
    cp = pltpu.async_copy(table.at[comb_v.at[pl.ds(off, TAIL)]], tail_v, g0)
    cp.wait()
    pltpu.sync_copy(tail_v, out.at[pl.ds(base + off, TAIL)])


def kernel(edge_attr, W_bond, W_stereo, W_conj):
    table = (W_bond[:, None, None, :]
             + W_stereo[None, :, None, :]
             + W_conj[None, None, :, :]).reshape(276, H)
    table = jnp.tile(table, (NW, 1))
    ea = edge_attr.astype(jnp.int32)
    e0, e1, e2 = ea[:, 0], ea[:, 1], ea[:, 2]
    mesh = plsc.VectorSubcoreMesh(core_axis_name="c", subcore_axis_name="s")
    run = functools.partial(
        pl.kernel,
        mesh=mesh,
        out_type=jax.ShapeDtypeStruct((E, H), jnp.float32),
        scratch_types=[
            pltpu.VMEM((PAD,), jnp.int32),
            pltpu.VMEM((PAD,), jnp.int32),
            pltpu.VMEM((NBUF, CHUNK, H), jnp.float32),
            pltpu.VMEM((TAIL, H), jnp.float32),
        ] + [pltpu.SemaphoreType.DMA] * (2 * NBUF),
    )(_sc_body)
    return run(e0, e1, e2, table)
